# Initial kernel scaffold; baseline (speedup 1.0000x reference)
import jax
import jax.numpy as jnp
from jax.experimental import pallas as pl


def kernel(x_nchw, conv1_w, conv1_b, conv2_w, conv2_b, fc1_w, fc1_b, fc2_w, fc2_b):
    raise NotImplementedError("write your pallas kernel here")


# speedup vs baseline: 30.2293x; 30.2293x over previous
import functools

import jax
import jax.numpy as jnp
from jax.experimental import pallas as pl
from jax.experimental.pallas import tpu as pltpu

H1 = W1 = 28
HP1 = WP1 = 30
HO1 = WO1 = 14
HP2 = WP2 = 16
C1 = 16
HO2 = WO2 = 7
C2 = 32
FC_IN = HO2 * WO2 * C2
FC_HID = 128
FC_OUT = 10

XLANES = 960
K1 = 96
N1 = 2 * 14 * C1
K2 = 3 * WP2 * C1
N2 = 2 * 7 * C2
Y1LANES = HP2 * WP2 * C1

_PARALLEL = pltpu.CompilerParams(dimension_semantics=("parallel",))


def _fused_cnn_kernel(x_ref, w1_ref, b1_ref, w2_ref, b2_ref,
                      fc1w_ref, fc1b_ref, fc2w_ref, fc2b_ref,
                      o_ref, y1_scr, y2_scr):
    y1_scr[:, pl.ds(0, WP2 * C1)] = jnp.zeros(
        (y1_scr.shape[0], WP2 * C1), jnp.float32)
    y1_scr[:, pl.ds(15 * WP2 * C1, WP2 * C1)] = jnp.zeros(
        (y1_scr.shape[0], WP2 * C1), jnp.float32)
    for r in range(1, 15):
        y1_scr[:, pl.ds(r * WP2 * C1, C1)] = jnp.zeros(
            (y1_scr.shape[0], C1), jnp.float32)
        y1_scr[:, pl.ds(r * WP2 * C1 + 15 * C1, C1)] = jnp.zeros(
            (y1_scr.shape[0], C1), jnp.float32)

    w1 = w1_ref[...]
    b1 = b1_ref[...]
    for i in range(HO1):
        y0 = jnp.dot(x_ref[:, pl.ds((2 * i) * WP1, K1)], w1,
                     preferred_element_type=jnp.float32)
        y1 = jnp.dot(x_ref[:, pl.ds((2 * i + 1) * WP1, K1)], w1,
                     preferred_element_type=jnp.float32)
        m = jnp.maximum(
            jnp.maximum(y0[:, :224], y0[:, 224:]),
            jnp.maximum(y1[:, :224], y1[:, 224:]))
        y1_scr[:, pl.ds((i + 1) * WP2 * C1 + C1, 14 * C1)] = (
            jnp.maximum(m + b1, 0.0))

    w2 = w2_ref[...]
    b2 = b2_ref[...]
    for i in range(HO2):
        z0 = jnp.dot(y1_scr[:, pl.ds((2 * i) * WP2 * C1, K2)], w2,
                     preferred_element_type=jnp.float32)
        z1 = jnp.dot(y1_scr[:, pl.ds((2 * i + 1) * WP2 * C1, K2)], w2,
                     preferred_element_type=jnp.float32)
        m = jnp.maximum(
            jnp.maximum(z0[:, :224], z0[:, 224:]),
            jnp.maximum(z1[:, :224], z1[:, 224:]))
        y2_scr[:, pl.ds(i * 7 * C2, 7 * C2)] = jnp.maximum(m + b2, 0.0)

    h = jnp.dot(y2_scr[...], fc1w_ref[...],
                preferred_element_type=jnp.float32)
    h = jnp.maximum(h + fc1b_ref[...], 0.0)
    o_ref[...] = (jnp.dot(h, fc2w_ref[...],
                          preferred_element_type=jnp.float32)
                  + fc2b_ref[...])


def _pick_block(batch):
    for cand in (512, 256, 128, 64, 32, 16, 8, 4, 2):
        if batch % cand == 0 and batch // cand >= 2:
            return cand
    return batch


def _build_w1(conv1_w):
    w4 = jnp.zeros((3, HP1, W1, C1), jnp.float32)
    idx = jnp.arange(W1)
    for dh in range(3):
        for dw in range(3):
            w4 = w4.at[dh, idx + dw, idx, :].add(conv1_w[dh * 3 + dw])
    w4 = w4.reshape(3, HP1, 14, 2, C1).transpose(0, 1, 3, 2, 4)
    w4 = w4.reshape(3 * HP1, N1)
    return jnp.pad(w4, ((0, K1 - 3 * HP1), (0, 0)))


def _build_w2(conv2_w):
    w5 = jnp.zeros((3, WP2, HO1, C1, C2), jnp.float32)
    idx = jnp.arange(HO1)
    for dh in range(3):
        for dw in range(3):
            t = dh * 3 + dw
            w5 = w5.at[dh, idx + dw, idx].add(
                conv2_w[t * C1:(t + 1) * C1, :][None])
    w5 = w5.transpose(0, 1, 3, 2, 4).reshape(K2, HO1, C2)
    w5 = w5.reshape(K2, 7, 2, C2).transpose(0, 2, 1, 3)
    return w5.reshape(K2, N2)


def kernel(x_nchw, conv1_w, conv1_b, conv2_w, conv2_b,
           fc1_w, fc1_b, fc2_w, fc2_b):
    batch = x_nchw.shape[0]
    bb = _pick_block(batch)

    xp = jnp.pad(x_nchw.reshape(batch, 28, 28).astype(jnp.float32),
                 ((0, 0), (1, 1), (1, 1)))
    xf = jnp.pad(xp.reshape(batch, HP1 * WP1),
                 ((0, 0), (0, XLANES - HP1 * WP1)))

    w1b = _build_w1(conv1_w)
    w2b = _build_w2(conv2_w)
    b1t = jnp.tile(conv1_b.reshape(1, C1), (1, 14))
    b2t = jnp.tile(conv2_b.reshape(1, C2), (1, 7))

    out = pl.pallas_call(
        _fused_cnn_kernel,
        out_shape=jax.ShapeDtypeStruct((batch, FC_HID), jnp.float32),
        grid=(batch // bb,),
        in_specs=[
            pl.BlockSpec((bb, XLANES), lambda i: (i, 0)),
            pl.BlockSpec((K1, N1), lambda i: (0, 0)),
            pl.BlockSpec((1, 224), lambda i: (0, 0)),
            pl.BlockSpec((K2, N2), lambda i: (0, 0)),
            pl.BlockSpec((1, 224), lambda i: (0, 0)),
            pl.BlockSpec((FC_IN, FC_HID), lambda i: (0, 0)),
            pl.BlockSpec((1, FC_HID), lambda i: (0, 0)),
            pl.BlockSpec((FC_HID, FC_HID), lambda i: (0, 0)),
            pl.BlockSpec((1, FC_HID), lambda i: (0, 0)),
        ],
        out_specs=pl.BlockSpec((bb, FC_HID), lambda i: (i, 0)),
        scratch_shapes=[
            pltpu.VMEM((bb, Y1LANES), jnp.float32),
            pltpu.VMEM((bb, FC_IN), jnp.float32),
        ],
        compiler_params=_PARALLEL,
    )(xf, w1b, b1t, w2b, b2t, fc1_w, fc1_b, fc2_w, fc2_b)
    return out[:, :FC_OUT]


# speedup vs baseline: 30.8219x; 1.0196x over previous
import jax
import jax.numpy as jnp
from jax.experimental import pallas as pl
from jax.experimental.pallas import tpu as pltpu

C1 = 16
C2 = 32
XLANES = 1024
N1 = 1024
K2 = 3 * 16 * C1
N2 = 512
Y1LANES = 16 * 16 * C1
Y2LANES = 7 * 8 * C2
FC_HID = 128
FC_OUT = 10

_PARALLEL = pltpu.CompilerParams(dimension_semantics=("parallel",))


def _fused_cnn_kernel(x_ref, w1e_ref, w1o_ref, b1_ref, w2_ref, b2_ref,
                      fc1w_ref, fc1b_ref, fc2w_ref, fc2b_ref,
                      o_ref, y1_scr, y2_scr):
    bb = x_ref.shape[0]
    y1_scr[:, pl.ds(0, 256)] = jnp.zeros((bb, 256), jnp.float32)
    y1_scr[:, pl.ds(15 * 256, 256)] = jnp.zeros((bb, 256), jnp.float32)

    w1e = w1e_ref[...]
    w1o = w1o_ref[...]
    b1 = b1_ref[...]
    for i in range(14):
        w = w1e if i % 2 == 0 else w1o
        y = jnp.dot(x_ref[:, pl.ds(128 * (i // 2), 256)], w,
                    preferred_element_type=jnp.float32)
        m = jnp.maximum(
            jnp.maximum(y[:, 0:256], y[:, 256:512]),
            jnp.maximum(y[:, 512:768], y[:, 768:1024]))
        y1_scr[:, pl.ds((i + 1) * 256, 256)] = jnp.maximum(m + b1, 0.0)

    w2 = w2_ref[...]
    b2 = b2_ref[...]
    for i in range(7):
        z0 = jnp.dot(y1_scr[:, pl.ds((2 * i) * 256, K2)], w2,
                     preferred_element_type=jnp.float32)
        z1 = jnp.dot(y1_scr[:, pl.ds((2 * i + 1) * 256, K2)], w2,
                     preferred_element_type=jnp.float32)
        m = jnp.maximum(
            jnp.maximum(z0[:, 0:256], z0[:, 256:512]),
            jnp.maximum(z1[:, 0:256], z1[:, 256:512]))
        y2_scr[:, pl.ds(i * 256, 256)] = jnp.maximum(m + b2, 0.0)

    h = jnp.dot(y2_scr[...], fc1w_ref[...],
                preferred_element_type=jnp.float32)
    h = jnp.maximum(h + fc1b_ref[...], 0.0)
    o_ref[...] = (jnp.dot(h, fc2w_ref[...],
                          preferred_element_type=jnp.float32)
                  + fc2b_ref[...])


def _pick_block(batch):
    for cand in (512, 256, 128, 64, 32, 16, 8, 4, 2):
        if batch % cand == 0 and batch // cand >= 2:
            return cand
    return batch


def _build_w1(conv1_w):
    w6 = jnp.zeros((8, 32, 2, 2, 16, C1), jnp.float32)
    j = jnp.arange(14)
    for dh in range(3):
        for dw in range(3):
            t = dh * 3 + dw
            for hp in range(2):
                for parity in range(2):
                    w_in = 2 * j + parity + dw
                    w6 = w6.at[hp + dh, w_in, hp, parity, j + 1, :].add(
                        conv1_w[t][None, :])
    w1e = w6.reshape(256, N1)
    w1o = jnp.roll(w1e, 64, axis=0)
    return w1e, w1o


def _build_w2(conv2_w):
    w7 = jnp.zeros((3, 16, C1, 2, 8, C2), jnp.float32)
    j = jnp.arange(7)
    for dh in range(3):
        for dw in range(3):
            t = dh * 3 + dw
            w2t = conv2_w[t * C1:(t + 1) * C1, :]
            for parity in range(2):
                w_in = 2 * j + parity + dw
                w7 = w7.at[dh, w_in, :, parity, j, :].add(w2t[None])
    return w7.reshape(K2, N2)


def kernel(x_nchw, conv1_w, conv1_b, conv2_w, conv2_b,
           fc1_w, fc1_b, fc2_w, fc2_b):
    batch = x_nchw.shape[0]
    bb = _pick_block(batch)

    xp = jnp.pad(x_nchw.reshape(batch, 28, 28).astype(jnp.float32),
                 ((0, 0), (1, 1), (1, 3)))
    xf = jnp.pad(xp.reshape(batch, 960), ((0, 0), (0, XLANES - 960)))

    w1e, w1o = _build_w1(conv1_w)
    w2b = _build_w2(conv2_w)
    b1t = jnp.zeros((16, C1), jnp.float32).at[1:15].set(
        jnp.broadcast_to(conv1_b.reshape(1, C1), (14, C1))).reshape(1, 256)
    b2t = jnp.zeros((8, C2), jnp.float32).at[:7].set(
        jnp.broadcast_to(conv2_b.reshape(1, C2), (7, C2))).reshape(1, 256)
    fc1p = jnp.pad(fc1_w.reshape(7, 7, C2, FC_HID),
                   ((0, 0), (0, 1), (0, 0), (0, 0))).reshape(Y2LANES, FC_HID)

    out = pl.pallas_call(
        _fused_cnn_kernel,
        out_shape=jax.ShapeDtypeStruct((batch, FC_HID), jnp.float32),
        grid=(batch // bb,),
        in_specs=[
            pl.BlockSpec((bb, XLANES), lambda i: (i, 0)),
            pl.BlockSpec((256, N1), lambda i: (0, 0)),
            pl.BlockSpec((256, N1), lambda i: (0, 0)),
            pl.BlockSpec((1, 256), lambda i: (0, 0)),
            pl.BlockSpec((K2, N2), lambda i: (0, 0)),
            pl.BlockSpec((1, 256), lambda i: (0, 0)),
            pl.BlockSpec((Y2LANES, FC_HID), lambda i: (0, 0)),
            pl.BlockSpec((1, FC_HID), lambda i: (0, 0)),
            pl.BlockSpec((FC_HID, FC_HID), lambda i: (0, 0)),
            pl.BlockSpec((1, FC_HID), lambda i: (0, 0)),
        ],
        out_specs=pl.BlockSpec((bb, FC_HID), lambda i: (i, 0)),
        scratch_shapes=[
            pltpu.VMEM((bb, Y1LANES), jnp.float32),
            pltpu.VMEM((bb, Y2LANES), jnp.float32),
        ],
        compiler_params=_PARALLEL,
    )(xf, w1e, w1o, b1t, w2b, b2t, fc1p, fc1_b, fc2_w, fc2_b)
    return out[:, :FC_OUT]


# speedup vs baseline: 39.4393x; 1.2796x over previous
import numpy as np

import jax
import jax.numpy as jnp
from jax.experimental import pallas as pl
from jax.experimental.pallas import tpu as pltpu

C1 = 16
C2 = 32
XLANES = 1024
N1 = 1024
K2 = 3 * 16 * C1
N2 = 512
Y1LANES = 16 * 16 * C1
Y2LANES = 7 * 8 * C2
FC_HID = 128
FC_OUT = 10

_PARALLEL = pltpu.CompilerParams(dimension_semantics=("parallel",))


def _fused_cnn_kernel(x_ref, w1e_ref, w1o_ref, b1_ref, w2_ref, b2_ref,
                      fc1w_ref, fc1b_ref, fc2w_ref, fc2b_ref,
                      o_ref, y1_scr, y2_scr):
    bb = x_ref.shape[0]
    y1_scr[:, pl.ds(0, 256)] = jnp.zeros((bb, 256), jnp.float32)
    y1_scr[:, pl.ds(15 * 256, 256)] = jnp.zeros((bb, 256), jnp.float32)

    w1e = w1e_ref[...]
    w1o = w1o_ref[...]
    b1 = b1_ref[...]
    for i in range(14):
        w = w1e if i % 2 == 0 else w1o
        y = jnp.dot(x_ref[:, pl.ds(128 * (i // 2), 256)], w,
                    preferred_element_type=jnp.float32)
        m = jnp.maximum(
            jnp.maximum(y[:, 0:256], y[:, 256:512]),
            jnp.maximum(y[:, 512:768], y[:, 768:1024]))
        y1_scr[:, pl.ds((i + 1) * 256, 256)] = jnp.maximum(m + b1, 0.0)

    w2 = w2_ref[...]
    b2 = b2_ref[...]
    for i in range(7):
        z0 = jnp.dot(y1_scr[:, pl.ds((2 * i) * 256, K2)], w2,
                     preferred_element_type=jnp.float32)
        z1 = jnp.dot(y1_scr[:, pl.ds((2 * i + 1) * 256, K2)], w2,
                     preferred_element_type=jnp.float32)
        m = jnp.maximum(
            jnp.maximum(z0[:, 0:256], z0[:, 256:512]),
            jnp.maximum(z1[:, 0:256], z1[:, 256:512]))
        y2_scr[:, pl.ds(i * 256, 256)] = jnp.maximum(m + b2, 0.0)

    h = jnp.dot(y2_scr[...], fc1w_ref[...],
                preferred_element_type=jnp.float32)
    h = jnp.maximum(h + fc1b_ref[...], 0.0)
    o_ref[...] = (jnp.dot(h, fc2w_ref[...],
                          preferred_element_type=jnp.float32)
                  + fc2b_ref[...])


def _pick_block(batch):
    for cand in (512, 256, 128, 64, 32, 16, 8, 4, 2):
        if batch % cand == 0 and batch // cand >= 2:
            return cand
    return batch


def _sel1():
    m = np.zeros((2, 8, 32, 2, 2, 16, 9), np.float32)
    for eo in range(2):
        for hp in range(2):
            for dh in range(3):
                r = hp + dh + 2 * eo
                for parity in range(2):
                    for jp in range(1, 15):
                        w_out = 2 * (jp - 1) + parity
                        for dw in range(3):
                            m[eo, r, w_out + dw, hp, parity, jp,
                              dh * 3 + dw] = 1.0
    return m.reshape(2 * 8 * 32 * 2 * 2 * 16, 9)


def _sel2():
    m = np.zeros((3, 16, C1, 2, 8, 9, C1), np.float32)
    for dh in range(3):
        for parity in range(2):
            for jp2 in range(7):
                w_out = 2 * jp2 + parity
                for dw in range(3):
                    for ci in range(C1):
                        m[dh, w_out + dw, ci, parity, jp2,
                          dh * 3 + dw, ci] = 1.0
    return m.reshape(3 * 16 * C1 * 2 * 8, 9 * C1)


_SEL1 = _sel1()
_SEL2 = _sel2()
_B1MASK = np.zeros((16, 1), np.float32)
_B1MASK[1:15] = 1.0
_B2MASK = np.zeros((8, 1), np.float32)
_B2MASK[:7] = 1.0


def kernel(x_nchw, conv1_w, conv1_b, conv2_w, conv2_b,
           fc1_w, fc1_b, fc2_w, fc2_b):
    batch = x_nchw.shape[0]
    bb = _pick_block(batch)

    xp = jnp.pad(x_nchw.reshape(batch, 28, 28).astype(jnp.float32),
                 ((0, 0), (1, 1), (1, 3)))
    xf = jnp.pad(xp.reshape(batch, 960), ((0, 0), (0, XLANES - 960)))

    w1pair = (jnp.asarray(_SEL1) @ conv1_w).reshape(2, 256, N1)
    w1e, w1o = w1pair[0], w1pair[1]
    w2b = (jnp.asarray(_SEL2) @ conv2_w).reshape(K2, N2)
    b1t = (jnp.asarray(_B1MASK) * conv1_b.reshape(1, C1)).reshape(1, 256)
    b2t = (jnp.asarray(_B2MASK) * conv2_b.reshape(1, C2)).reshape(1, 256)
    fc1p = jnp.pad(fc1_w.reshape(7, 7, C2, FC_HID),
                   ((0, 0), (0, 1), (0, 0), (0, 0))).reshape(Y2LANES, FC_HID)

    out = pl.pallas_call(
        _fused_cnn_kernel,
        out_shape=jax.ShapeDtypeStruct((batch, FC_HID), jnp.float32),
        grid=(batch // bb,),
        in_specs=[
            pl.BlockSpec((bb, XLANES), lambda i: (i, 0)),
            pl.BlockSpec((256, N1), lambda i: (0, 0)),
            pl.BlockSpec((256, N1), lambda i: (0, 0)),
            pl.BlockSpec((1, 256), lambda i: (0, 0)),
            pl.BlockSpec((K2, N2), lambda i: (0, 0)),
            pl.BlockSpec((1, 256), lambda i: (0, 0)),
            pl.BlockSpec((Y2LANES, FC_HID), lambda i: (0, 0)),
            pl.BlockSpec((1, FC_HID), lambda i: (0, 0)),
            pl.BlockSpec((FC_HID, FC_HID), lambda i: (0, 0)),
            pl.BlockSpec((1, FC_HID), lambda i: (0, 0)),
        ],
        out_specs=pl.BlockSpec((bb, FC_HID), lambda i: (i, 0)),
        scratch_shapes=[
            pltpu.VMEM((bb, Y1LANES), jnp.float32),
            pltpu.VMEM((bb, Y2LANES), jnp.float32),
        ],
        compiler_params=_PARALLEL,
    )(xf, w1e, w1o, b1t, w2b, b2t, fc1p, fc1_b, fc2_w, fc2_b)
    return out[:, :FC_OUT]
